# baseline (speedup 1.0000x reference)
import functools

import jax
import jax.numpy as jnp
from jax import lax
from jax.experimental import pallas as pl
from jax.experimental.pallas import tpu as pltpu
from jax.experimental.pallas import tpu_sc as plsc

NUM_CORES = 2
NUM_SUBCORES = 16
NUM_WORKERS = NUM_CORES * NUM_SUBCORES
LANES = 16

GS = 16
RB = 6400
NGB = RB // GS
CHS = 32
CHE = 16
BP = 1024


def _bk_body(B, N, gl_ref, p1_ref, p2_ref, p3_ref, wdn_ref, wlab_ref,
             wdff_ref, off_ref, wb_ref, jm_ref, eff_ref):
    gl8 = gl_ref[...]
    glf = gl8.astype(jnp.float32)

    io_r = lax.broadcasted_iota(jnp.int32, (128, 128), 0)
    io_c = lax.broadcasted_iota(jnp.int32, (128, 128), 1)
    tri = (io_r <= io_c).astype(jnp.float32)
    s1 = jnp.dot(glf, tri,
                 precision=lax.Precision.HIGHEST)
    rowtot = s1[:, 127:128]

    def shift(x, k):
        return jnp.concatenate(
            [jnp.zeros((k, 1), jnp.float32), x[:8 - k, :]], axis=0)

    s = rowtot
    s = s + shift(s, 1)
    s = s + shift(s, 2)
    s = s + shift(s, 4)
    rowpre = s - rowtot

    off_i = (s1 + rowpre).astype(jnp.int32)
    off_ref[...] = off_i

    ends8 = jnp.minimum(off_i, N)
    totalr = jnp.max(ends8)

    lane48 = lax.broadcasted_iota(jnp.int32, (1, 48), 1)
    acc = jnp.where(lane48 == NUM_WORKERS, B, 0)
    for w in range(1, NUM_WORKERS):
        tw = (w * totalr) >> 5
        cnt = jnp.sum((ends8 < tw).astype(jnp.int32))
        acc = acc + jnp.where(lane48 == w, cnt, 0)
    wb_ref[...] = acc

    jm = jnp.maximum((totalr + RB - 1) // RB, 1)
    jm_ref[...] = jnp.reshape(jm, (1, 1))

    wl0 = wlab_ref[0]
    wl1 = wlab_ref[1]
    wl2 = wlab_ref[2]
    wd0 = wdff_ref[0]
    wd1 = wdff_ref[1]
    eff_ref[...] = (wd0 * (1.0 + wl0 * p1_ref[...] + wl1 * p2_ref[...]
                           + wl2 * p3_ref[...]) + wd1 * wdn_ref[...])


def _tc_body(jmax, seqa_ref, seqb_ref, eff_ref, gact_ref):
    j = pl.program_id(0)
    eff = eff_ref[...]

    def emit(src_ref):
        t = eff * src_ref[...]
        act = jnp.where(t > 0.0, t, jnp.exp(t) - 1.0)
        gact_ref[...] = act.reshape(NGB, GS, act.shape[1]).sum(axis=1)

    @pl.when(j & 1 == 0)
    def _():
        emit(seqa_ref)

    @pl.when(j & 1 == 1)
    def _():
        emit(seqb_ref)


def _sc_body(N, F, B, NGT, seq_h, gact_h, off_h, wb_h, eff_h,
             out_h, gb0_v, lb0_v, hb0_v, gb1_v, lb1_v, hb1_v,
             off_v, wbv_v, eff_v, row_v, sem_a, sem_b):
    nj = F // LANES
    CW = CHS * F

    cid = lax.axis_index("c")
    sid = lax.axis_index("s")
    wid = sid * NUM_CORES + cid

    pltpu.sync_copy(off_h, off_v)
    pltpu.sync_copy(wb_h, wbv_v)
    pltpu.sync_copy(eff_h, eff_v)

    effs = tuple(eff_v[pl.ds(j * LANES, LANES)] for j in range(nj))
    io16 = jnp.arange(LANES, dtype=jnp.int32)

    def pick(v, i):
        r = v[8]
        for q in range(7, -1, -1):
            r = jnp.where(i == q, v[q], r)
        return r

    wa = (wid >> 3) << 3
    vw = wbv_v[pl.ds(wa, LANES)]
    lo = pick(vw, wid - wa)
    hi = pick(vw, wid + 1 - wa)

    def seg_params(k):
        km = jnp.maximum(k - 1, 0)
        a = (km >> 3) << 3
        v = off_v[pl.ds(a, LANES)]
        sp = jnp.where(k == 0, 0, pick(v, km - a))
        s = jnp.minimum(sp, N)
        e = jnp.minimum(pick(v, k - a), N)
        ln = e - s
        ga = (s + (GS - 1)) >> 4
        gb = e >> 4
        ng = jnp.maximum(gb - ga, 0)
        locnt = jnp.where(gb > ga, ga * GS - s, ln)
        hicnt = jnp.where(gb > ga, e - gb * GS, 0)
        return s, e, ga, gb, ng, locnt, hicnt

    def fire(k, gb_v, lb_v, hb_v, sem):
        s, e, ga, gb, ng, locnt, hicnt = seg_params(k)

        @pl.when(ng > 0)
        def _():
            ag = jnp.minimum(ga, NGT - CHS)
            pltpu.async_copy(gact_h.at[pl.ds(ag * F, CW)], gb_v, sem)

        @pl.when(locnt > 0)
        def _():
            al = jnp.minimum(s, N - CHS)
            pltpu.async_copy(seq_h.at[pl.ds(al * F, CW)], lb_v, sem)

        @pl.when(hicnt > 0)
        def _():
            ah = jnp.minimum(gb * GS, N - CHE)
            pltpu.async_copy(seq_h.at[pl.ds(ah * F, CHE * F)], hb_v, sem)

    def drain_compute_write(k, gb_v, lb_v, hb_v, sem, b):
        s, e, ga, gb, ng, locnt, hicnt = seg_params(k)

        @pl.when(ng > 0)
        def _():
            pltpu.make_async_copy(gact_h.at[pl.ds(0, CW)], gb_v, sem).wait()

        @pl.when(locnt > 0)
        def _():
            pltpu.make_async_copy(seq_h.at[pl.ds(0, CW)], lb_v, sem).wait()

        @pl.when(hicnt > 0)
        def _():
            pltpu.make_async_copy(seq_h.at[pl.ds(0, CHE * F)], hb_v,
                                  sem).wait()

        zeros = tuple(jnp.zeros((LANES,), jnp.float32) for _ in range(nj))

        dg = ga - jnp.minimum(ga, NGT - CHS)

        def g_body(i, accs):
            off = i * F
            return tuple(accs[j] + gb_v[pl.ds(off + j * LANES, LANES)]
                         for j in range(nj))

        accs = lax.fori_loop(dg, dg + ng, g_body, zeros)

        def edge_body(buf):
            def body(i, accs):
                off = i * F
                new = []
                for j in range(nj):
                    x = buf[pl.ds(off + j * LANES, LANES)]
                    t = effs[j] * x
                    y = jnp.where(t > 0.0, t, jnp.exp(t) - 1.0)
                    new.append(accs[j] + y)
                return tuple(new)
            return body

        dl = s - jnp.minimum(s, N - CHS)
        accs = lax.fori_loop(dl, dl + locnt, edge_body(lb_v), accs)
        dh = gb * GS - jnp.minimum(gb * GS, N - CHE)
        accs = lax.fori_loop(dh, dh + hicnt, edge_body(hb_v), accs)

        for j in range(nj):
            row_v[pl.ds(j * LANES, LANES)] = accs[j]
        pltpu.sync_copy(row_v, out_h.at[pl.ds(b * F, F)])

    @pl.when(lo < hi)
    def _():
        fire(lo, gb0_v, lb0_v, hb0_v, sem_a)

    npairs = (hi - lo + 1) >> 1

    def pair_body(kk, carry):
        k0 = lo + 2 * kk
        k1 = k0 + 1

        @pl.when(k1 < hi)
        def _():
            fire(k1, gb1_v, lb1_v, hb1_v, sem_b)

        drain_compute_write(k0, gb0_v, lb0_v, hb0_v, sem_a, k0)

        @pl.when(k0 + 2 < hi)
        def _():
            fire(k0 + 2, gb0_v, lb0_v, hb0_v, sem_a)

        @pl.when(k1 < hi)
        def _():
            drain_compute_write(k1, gb1_v, lb1_v, hb1_v, sem_b, k1)

        return carry

    lax.fori_loop(0, npairs, pair_body, 0)


def kernel(seq, graph_len, prompt1, prompt2, prompt3, w_label, w_dff, w_down):
    N, F = seq.shape
    B = graph_len.shape[0]
    NB = N // RB
    NGT = N // GS

    gl8 = jnp.concatenate(
        [graph_len.astype(jnp.int32),
         jnp.zeros((BP - B,), jnp.int32)]).reshape(8, BP // 8)

    off8, wb48, jm, eff = pl.pallas_call(
        functools.partial(_bk_body, B, N),
        grid=(1,),
        in_specs=[
            pl.BlockSpec((8, BP // 8), lambda i: (0, 0)),
            pl.BlockSpec((1, F), lambda i: (0, 0)),
            pl.BlockSpec((1, F), lambda i: (0, 0)),
            pl.BlockSpec((1, F), lambda i: (0, 0)),
            pl.BlockSpec((1, F), lambda i: (0, 0)),
            pl.BlockSpec(memory_space=pltpu.SMEM),
            pl.BlockSpec(memory_space=pltpu.SMEM),
        ],
        out_specs=[
            pl.BlockSpec((8, BP // 8), lambda i: (0, 0)),
            pl.BlockSpec((1, 48), lambda i: (0, 0)),
            pl.BlockSpec((1, 1), lambda i: (0, 0)),
            pl.BlockSpec((1, F), lambda i: (0, 0)),
        ],
        out_shape=[
            jax.ShapeDtypeStruct((8, BP // 8), jnp.int32),
            jax.ShapeDtypeStruct((1, 48), jnp.int32),
            jax.ShapeDtypeStruct((1, 1), jnp.int32),
            jax.ShapeDtypeStruct((1, F), jnp.float32),
        ],
    )(gl8, prompt1, prompt2, prompt3, w_down,
      w_label.reshape(-1), w_dff.reshape(-1))

    gact = pl.pallas_call(
        _tc_body,
        grid_spec=pltpu.PrefetchScalarGridSpec(
            num_scalar_prefetch=1,
            grid=(NB,),
            in_specs=[
                pl.BlockSpec(
                    (RB, F),
                    lambda j, jm: (jnp.minimum(2 * (j // 2), jm[0] - 1), 0)),
                pl.BlockSpec(
                    (RB, F),
                    lambda j, jm: (jnp.minimum(2 * (j // 2) + 1, jm[0] - 1), 0)),
                pl.BlockSpec((1, F), lambda j, jm: (0, 0)),
            ],
            out_specs=pl.BlockSpec(
                (NGB, F), lambda j, jm: (jnp.minimum(j, jm[0] - 1), 0)),
        ),
        out_shape=jax.ShapeDtypeStruct((NGT, F), jnp.float32),
    )(jm.reshape(1), seq, seq, eff)

    mesh = plsc.VectorSubcoreMesh(core_axis_name="c", subcore_axis_name="s",
                                  num_cores=NUM_CORES,
                                  num_subcores=NUM_SUBCORES)
    body = functools.partial(_sc_body, N, F, B, NGT)
    out_flat = pl.kernel(
        body,
        out_type=jax.ShapeDtypeStruct((B * F,), jnp.float32),
        mesh=mesh,
        scratch_types=[
            pltpu.VMEM((CHS * F,), jnp.float32),
            pltpu.VMEM((CHS * F,), jnp.float32),
            pltpu.VMEM((CHE * F,), jnp.float32),
            pltpu.VMEM((CHS * F,), jnp.float32),
            pltpu.VMEM((CHS * F,), jnp.float32),
            pltpu.VMEM((CHE * F,), jnp.float32),
            pltpu.VMEM((BP,), jnp.int32),
            pltpu.VMEM((48,), jnp.int32),
            pltpu.VMEM((F,), jnp.float32),
            pltpu.VMEM((F,), jnp.float32),
            pltpu.SemaphoreType.DMA,
            pltpu.SemaphoreType.DMA,
        ],
    )(seq.reshape(-1), gact.reshape(-1), off8.reshape(-1), wb48.reshape(-1),
      eff.reshape(-1))
    return out_flat.reshape(B, F)


# speedup vs baseline: 1.1889x; 1.1889x over previous
import functools

import jax
import jax.numpy as jnp
from jax import lax
from jax.experimental import pallas as pl
from jax.experimental.pallas import tpu as pltpu
from jax.experimental.pallas import tpu_sc as plsc

NUM_CORES = 2
NUM_SUBCORES = 16
NUM_WORKERS = NUM_CORES * NUM_SUBCORES
LANES = 16

GS = 16
RB = 6400
NGB = RB // GS
CHS = 32
CHE = 16
BP = 1024


def _bk_body(B, N, gl_ref, p1_ref, p2_ref, p3_ref, wdn_ref, wlab_ref,
             wdff_ref, off_ref, wb_ref, jm_ref, eff_ref):
    gl8 = gl_ref[...]
    glf = gl8.astype(jnp.float32)

    io_r = lax.broadcasted_iota(jnp.int32, (128, 128), 0)
    io_c = lax.broadcasted_iota(jnp.int32, (128, 128), 1)
    tri = (io_r <= io_c).astype(jnp.float32)
    s1 = jnp.dot(glf, tri,
                 precision=lax.Precision.HIGHEST)
    rowtot = s1[:, 127:128]

    def shift(x, k):
        return jnp.concatenate(
            [jnp.zeros((k, 1), jnp.float32), x[:8 - k, :]], axis=0)

    s = rowtot
    s = s + shift(s, 1)
    s = s + shift(s, 2)
    s = s + shift(s, 4)
    rowpre = s - rowtot

    off_i = (s1 + rowpre).astype(jnp.int32)
    off_ref[...] = off_i

    ends8 = jnp.minimum(off_i, N)
    totalr = jnp.max(ends8)

    lane48 = lax.broadcasted_iota(jnp.int32, (1, 48), 1)
    acc = jnp.where(lane48 == NUM_WORKERS, B, 0)
    for w in range(1, NUM_WORKERS):
        tw = (w * totalr) >> 5
        cnt = jnp.sum((ends8 < tw).astype(jnp.int32))
        acc = acc + jnp.where(lane48 == w, cnt, 0)
    wb_ref[...] = acc

    jm = jnp.maximum((totalr + RB - 1) // RB, 1)
    jm_ref[...] = jnp.reshape(jm, (1, 1))

    wl0 = wlab_ref[0]
    wl1 = wlab_ref[1]
    wl2 = wlab_ref[2]
    wd0 = wdff_ref[0]
    wd1 = wdff_ref[1]
    eff_ref[...] = (wd0 * (1.0 + wl0 * p1_ref[...] + wl1 * p2_ref[...]
                           + wl2 * p3_ref[...]) + wd1 * wdn_ref[...])


def _tc_body(jmax, seq_ref, eff_ref, gact_ref):
    t = eff_ref[...] * seq_ref[...]
    act = jnp.where(t > 0.0, t, jnp.exp(t) - 1.0)
    gact_ref[...] = act.reshape(NGB, GS, act.shape[1]).sum(axis=1)


def _sc_body(N, F, B, NGT, seq_h, gact_h, off_h, wb_h, eff_h,
             out_h, gb0_v, lb0_v, hb0_v, gb1_v, lb1_v, hb1_v,
             off_v, wbv_v, eff_v, row_v, sem_a, sem_b):
    nj = F // LANES
    CW = CHS * F

    cid = lax.axis_index("c")
    sid = lax.axis_index("s")
    wid = sid * NUM_CORES + cid

    pltpu.sync_copy(off_h, off_v)
    pltpu.sync_copy(wb_h, wbv_v)
    pltpu.sync_copy(eff_h, eff_v)

    effs = tuple(eff_v[pl.ds(j * LANES, LANES)] for j in range(nj))
    io16 = jnp.arange(LANES, dtype=jnp.int32)

    def pick(v, i):
        r = v[8]
        for q in range(7, -1, -1):
            r = jnp.where(i == q, v[q], r)
        return r

    wa = (wid >> 3) << 3
    vw = wbv_v[pl.ds(wa, LANES)]
    lo = pick(vw, wid - wa)
    hi = pick(vw, wid + 1 - wa)

    def seg_params(k):
        km = jnp.maximum(k - 1, 0)
        a = (km >> 3) << 3
        v = off_v[pl.ds(a, LANES)]
        sp = jnp.where(k == 0, 0, pick(v, km - a))
        s = jnp.minimum(sp, N)
        e = jnp.minimum(pick(v, k - a), N)
        ln = e - s
        ga = (s + (GS - 1)) >> 4
        gb = e >> 4
        ng = jnp.maximum(gb - ga, 0)
        locnt = jnp.where(gb > ga, ga * GS - s, ln)
        hicnt = jnp.where(gb > ga, e - gb * GS, 0)
        return s, e, ga, gb, ng, locnt, hicnt

    HW = CHE * F

    def fire(k, gb_v, lb_v, hb_v, sem):
        s, e, ga, gb, ng, locnt, hicnt = seg_params(k)

        ag = jnp.minimum(ga, NGT - CHS)
        dg = ga - ag

        @pl.when(ng > 0)
        def _():
            pltpu.async_copy(gact_h.at[pl.ds(ag * F, HW)],
                             gb_v.at[pl.ds(0, HW)], sem)

        @pl.when(dg + ng > CHE)
        def _():
            pltpu.async_copy(gact_h.at[pl.ds((ag + CHE) * F, HW)],
                             gb_v.at[pl.ds(HW, HW)], sem)

        al = jnp.minimum(s, N - CHS)
        dl = s - al

        @pl.when(locnt > 0)
        def _():
            pltpu.async_copy(seq_h.at[pl.ds(al * F, HW)],
                             lb_v.at[pl.ds(0, HW)], sem)

        @pl.when(dl + locnt > CHE)
        def _():
            pltpu.async_copy(seq_h.at[pl.ds((al + CHE) * F, HW)],
                             lb_v.at[pl.ds(HW, HW)], sem)

        @pl.when(hicnt > 0)
        def _():
            ah = jnp.minimum(gb * GS, N - CHE)
            pltpu.async_copy(seq_h.at[pl.ds(ah * F, CHE * F)], hb_v, sem)

    def drain_compute_write(k, gb_v, lb_v, hb_v, sem, b):
        s, e, ga, gb, ng, locnt, hicnt = seg_params(k)

        dg = ga - jnp.minimum(ga, NGT - CHS)
        dl0 = s - jnp.minimum(s, N - CHS)

        @pl.when(ng > 0)
        def _():
            pltpu.make_async_copy(gact_h.at[pl.ds(0, HW)],
                                  gb_v.at[pl.ds(0, HW)], sem).wait()

        @pl.when(dg + ng > CHE)
        def _():
            pltpu.make_async_copy(gact_h.at[pl.ds(0, HW)],
                                  gb_v.at[pl.ds(HW, HW)], sem).wait()

        @pl.when(locnt > 0)
        def _():
            pltpu.make_async_copy(seq_h.at[pl.ds(0, HW)],
                                  lb_v.at[pl.ds(0, HW)], sem).wait()

        @pl.when(dl0 + locnt > CHE)
        def _():
            pltpu.make_async_copy(seq_h.at[pl.ds(0, HW)],
                                  lb_v.at[pl.ds(HW, HW)], sem).wait()

        @pl.when(hicnt > 0)
        def _():
            pltpu.make_async_copy(seq_h.at[pl.ds(0, CHE * F)], hb_v,
                                  sem).wait()

        zeros = tuple(jnp.zeros((LANES,), jnp.float32) for _ in range(nj))


        def g_body(i, accs):
            off = i * F
            return tuple(accs[j] + gb_v[pl.ds(off + j * LANES, LANES)]
                         for j in range(nj))

        accs = lax.fori_loop(dg, dg + ng, g_body, zeros)

        def edge_body(buf):
            def body(i, accs):
                off = i * F
                new = []
                for j in range(nj):
                    x = buf[pl.ds(off + j * LANES, LANES)]
                    t = effs[j] * x
                    y = jnp.where(t > 0.0, t, jnp.exp(t) - 1.0)
                    new.append(accs[j] + y)
                return tuple(new)
            return body

        dl = s - jnp.minimum(s, N - CHS)
        accs = lax.fori_loop(dl, dl + locnt, edge_body(lb_v), accs)
        dh = gb * GS - jnp.minimum(gb * GS, N - CHE)
        accs = lax.fori_loop(dh, dh + hicnt, edge_body(hb_v), accs)

        for j in range(nj):
            row_v[pl.ds(j * LANES, LANES)] = accs[j]
        pltpu.sync_copy(row_v, out_h.at[pl.ds(b * F, F)])

    @pl.when(lo < hi)
    def _():
        fire(lo, gb0_v, lb0_v, hb0_v, sem_a)

    npairs = (hi - lo + 1) >> 1

    def pair_body(kk, carry):
        k0 = lo + 2 * kk
        k1 = k0 + 1

        @pl.when(k1 < hi)
        def _():
            fire(k1, gb1_v, lb1_v, hb1_v, sem_b)

        drain_compute_write(k0, gb0_v, lb0_v, hb0_v, sem_a, k0)

        @pl.when(k0 + 2 < hi)
        def _():
            fire(k0 + 2, gb0_v, lb0_v, hb0_v, sem_a)

        @pl.when(k1 < hi)
        def _():
            drain_compute_write(k1, gb1_v, lb1_v, hb1_v, sem_b, k1)

        return carry

    lax.fori_loop(0, npairs, pair_body, 0)


def kernel(seq, graph_len, prompt1, prompt2, prompt3, w_label, w_dff, w_down):
    N, F = seq.shape
    B = graph_len.shape[0]
    NB = N // RB
    NGT = N // GS

    gl8 = jnp.concatenate(
        [graph_len.astype(jnp.int32),
         jnp.zeros((BP - B,), jnp.int32)]).reshape(8, BP // 8)

    off8, wb48, jm, eff = pl.pallas_call(
        functools.partial(_bk_body, B, N),
        grid=(1,),
        in_specs=[
            pl.BlockSpec((8, BP // 8), lambda i: (0, 0)),
            pl.BlockSpec((1, F), lambda i: (0, 0)),
            pl.BlockSpec((1, F), lambda i: (0, 0)),
            pl.BlockSpec((1, F), lambda i: (0, 0)),
            pl.BlockSpec((1, F), lambda i: (0, 0)),
            pl.BlockSpec(memory_space=pltpu.SMEM),
            pl.BlockSpec(memory_space=pltpu.SMEM),
        ],
        out_specs=[
            pl.BlockSpec((8, BP // 8), lambda i: (0, 0)),
            pl.BlockSpec((1, 48), lambda i: (0, 0)),
            pl.BlockSpec((1, 1), lambda i: (0, 0)),
            pl.BlockSpec((1, F), lambda i: (0, 0)),
        ],
        out_shape=[
            jax.ShapeDtypeStruct((8, BP // 8), jnp.int32),
            jax.ShapeDtypeStruct((1, 48), jnp.int32),
            jax.ShapeDtypeStruct((1, 1), jnp.int32),
            jax.ShapeDtypeStruct((1, F), jnp.float32),
        ],
    )(gl8, prompt1, prompt2, prompt3, w_down,
      w_label.reshape(-1), w_dff.reshape(-1))

    gact = pl.pallas_call(
        _tc_body,
        grid_spec=pltpu.PrefetchScalarGridSpec(
            num_scalar_prefetch=1,
            grid=(NB,),
            in_specs=[
                pl.BlockSpec((RB, F), lambda j, jm: (jnp.minimum(j, jm[0] - 1), 0)),
                pl.BlockSpec((1, F), lambda j, jm: (0, 0)),
            ],
            out_specs=pl.BlockSpec(
                (NGB, F), lambda j, jm: (jnp.minimum(j, jm[0] - 1), 0)),
        ),
        out_shape=jax.ShapeDtypeStruct((NGT, F), jnp.float32),
    )(jm.reshape(1), seq, eff)

    mesh = plsc.VectorSubcoreMesh(core_axis_name="c", subcore_axis_name="s",
                                  num_cores=NUM_CORES,
                                  num_subcores=NUM_SUBCORES)
    body = functools.partial(_sc_body, N, F, B, NGT)
    out_flat = pl.kernel(
        body,
        out_type=jax.ShapeDtypeStruct((B * F,), jnp.float32),
        mesh=mesh,
        scratch_types=[
            pltpu.VMEM((CHS * F,), jnp.float32),
            pltpu.VMEM((CHS * F,), jnp.float32),
            pltpu.VMEM((CHE * F,), jnp.float32),
            pltpu.VMEM((CHS * F,), jnp.float32),
            pltpu.VMEM((CHS * F,), jnp.float32),
            pltpu.VMEM((CHE * F,), jnp.float32),
            pltpu.VMEM((BP,), jnp.int32),
            pltpu.VMEM((48,), jnp.int32),
            pltpu.VMEM((F,), jnp.float32),
            pltpu.VMEM((F,), jnp.float32),
            pltpu.SemaphoreType.DMA,
            pltpu.SemaphoreType.DMA,
        ],
    )(seq.reshape(-1), gact.reshape(-1), off8.reshape(-1), wb48.reshape(-1),
      eff.reshape(-1))
    return out_flat.reshape(B, F)


# speedup vs baseline: 1.4240x; 1.1977x over previous
import functools

import jax
import jax.numpy as jnp
from jax import lax
from jax.experimental import pallas as pl
from jax.experimental.pallas import tpu as pltpu
from jax.experimental.pallas import tpu_sc as plsc

NUM_CORES = 2
NUM_SUBCORES = 16
NUM_WORKERS = NUM_CORES * NUM_SUBCORES
LANES = 16

GS = 16
RB = 12800
NGB = RB // GS
CHS = 32
CHE = 16
BP = 1024


def _bk_body(B, N, gl_ref, p1_ref, p2_ref, p3_ref, wdn_ref, wlab_ref,
             wdff_ref, off_ref, wb_ref, jm_ref, eff_ref):
    gl8 = gl_ref[...]
    glf = gl8.astype(jnp.float32)

    io_r = lax.broadcasted_iota(jnp.int32, (128, 128), 0)
    io_c = lax.broadcasted_iota(jnp.int32, (128, 128), 1)
    tri = (io_r <= io_c).astype(jnp.float32)
    s1 = jnp.dot(glf, tri,
                 precision=lax.Precision.HIGHEST)
    rowtot = s1[:, 127:128]

    def shift(x, k):
        return jnp.concatenate(
            [jnp.zeros((k, 1), jnp.float32), x[:8 - k, :]], axis=0)

    s = rowtot
    s = s + shift(s, 1)
    s = s + shift(s, 2)
    s = s + shift(s, 4)
    rowpre = s - rowtot

    off_i = (s1 + rowpre).astype(jnp.int32)
    off_ref[...] = off_i

    ends8 = jnp.minimum(off_i, N)
    totalr = jnp.max(ends8)

    lane48 = lax.broadcasted_iota(jnp.int32, (1, 48), 1)
    acc = jnp.where(lane48 == NUM_WORKERS, B, 0)
    for w in range(1, NUM_WORKERS):
        tw = (w * totalr) >> 5
        cnt = jnp.sum((ends8 < tw).astype(jnp.int32))
        acc = acc + jnp.where(lane48 == w, cnt, 0)
    wb_ref[...] = acc

    jm = jnp.maximum((totalr + RB - 1) // RB, 1)
    jm_ref[...] = jnp.reshape(jm, (1, 1))

    wl0 = wlab_ref[0]
    wl1 = wlab_ref[1]
    wl2 = wlab_ref[2]
    wd0 = wdff_ref[0]
    wd1 = wdff_ref[1]
    eff_ref[...] = (wd0 * (1.0 + wl0 * p1_ref[...] + wl1 * p2_ref[...]
                           + wl2 * p3_ref[...]) + wd1 * wdn_ref[...])


def _tc_body(jmax, seq_ref, eff_ref, gact_ref):
    @pl.when(pl.program_id(0) < jmax[0])
    def _():
        t = eff_ref[...] * seq_ref[...]
        act = jnp.where(t > 0.0, t, jnp.exp(t) - 1.0)
        gact_ref[...] = act.reshape(NGB, GS, act.shape[1]).sum(axis=1)


def _sc_body(N, F, B, NGT, seq_h, gact_h, off_h, wb_h, eff_h,
             out_h, gb0_v, lb0_v, hb0_v, gb1_v, lb1_v, hb1_v,
             off_v, wbv_v, eff_v, row_v, sem_a, sem_b):
    nj = F // LANES
    CW = CHS * F

    cid = lax.axis_index("c")
    sid = lax.axis_index("s")
    wid = sid * NUM_CORES + cid

    pltpu.sync_copy(off_h, off_v)
    pltpu.sync_copy(wb_h, wbv_v)
    pltpu.sync_copy(eff_h, eff_v)

    effs = tuple(eff_v[pl.ds(j * LANES, LANES)] for j in range(nj))
    io16 = jnp.arange(LANES, dtype=jnp.int32)

    def pick(v, i):
        r = v[8]
        for q in range(7, -1, -1):
            r = jnp.where(i == q, v[q], r)
        return r

    wa = (wid >> 3) << 3
    vw = wbv_v[pl.ds(wa, LANES)]
    lo = pick(vw, wid - wa)
    hi = pick(vw, wid + 1 - wa)

    def seg_params(k):
        km = jnp.maximum(k - 1, 0)
        a = (km >> 3) << 3
        v = off_v[pl.ds(a, LANES)]
        sp = jnp.where(k == 0, 0, pick(v, km - a))
        s = jnp.minimum(sp, N)
        e = jnp.minimum(pick(v, k - a), N)
        ln = e - s
        ga = (s + (GS - 1)) >> 4
        gb = e >> 4
        ng = jnp.maximum(gb - ga, 0)
        locnt = jnp.where(gb > ga, ga * GS - s, ln)
        hicnt = jnp.where(gb > ga, e - gb * GS, 0)
        return s, e, ga, gb, ng, locnt, hicnt

    HW = CHE * F

    def fire(k, gb_v, lb_v, hb_v, sem):
        s, e, ga, gb, ng, locnt, hicnt = seg_params(k)

        ag = jnp.minimum(ga, NGT - CHS)
        dg = ga - ag

        @pl.when(ng > 0)
        def _():
            pltpu.async_copy(gact_h.at[pl.ds(ag * F, HW)],
                             gb_v.at[pl.ds(0, HW)], sem)

        @pl.when(dg + ng > CHE)
        def _():
            pltpu.async_copy(gact_h.at[pl.ds((ag + CHE) * F, HW)],
                             gb_v.at[pl.ds(HW, HW)], sem)

        al = jnp.minimum(s, N - CHS)
        dl = s - al

        @pl.when(locnt > 0)
        def _():
            pltpu.async_copy(seq_h.at[pl.ds(al * F, HW)],
                             lb_v.at[pl.ds(0, HW)], sem)

        @pl.when(dl + locnt > CHE)
        def _():
            pltpu.async_copy(seq_h.at[pl.ds((al + CHE) * F, HW)],
                             lb_v.at[pl.ds(HW, HW)], sem)

        @pl.when(hicnt > 0)
        def _():
            ah = jnp.minimum(gb * GS, N - CHE)
            pltpu.async_copy(seq_h.at[pl.ds(ah * F, CHE * F)], hb_v, sem)

    def drain_compute_write(k, gb_v, lb_v, hb_v, sem, b):
        s, e, ga, gb, ng, locnt, hicnt = seg_params(k)

        dg = ga - jnp.minimum(ga, NGT - CHS)
        dl0 = s - jnp.minimum(s, N - CHS)

        @pl.when(ng > 0)
        def _():
            pltpu.make_async_copy(gact_h.at[pl.ds(0, HW)],
                                  gb_v.at[pl.ds(0, HW)], sem).wait()

        @pl.when(dg + ng > CHE)
        def _():
            pltpu.make_async_copy(gact_h.at[pl.ds(0, HW)],
                                  gb_v.at[pl.ds(HW, HW)], sem).wait()

        @pl.when(locnt > 0)
        def _():
            pltpu.make_async_copy(seq_h.at[pl.ds(0, HW)],
                                  lb_v.at[pl.ds(0, HW)], sem).wait()

        @pl.when(dl0 + locnt > CHE)
        def _():
            pltpu.make_async_copy(seq_h.at[pl.ds(0, HW)],
                                  lb_v.at[pl.ds(HW, HW)], sem).wait()

        @pl.when(hicnt > 0)
        def _():
            pltpu.make_async_copy(seq_h.at[pl.ds(0, CHE * F)], hb_v,
                                  sem).wait()

        zeros = tuple(jnp.zeros((LANES,), jnp.float32) for _ in range(nj))


        def g_body(i, accs):
            off = i * F
            return tuple(accs[j] + gb_v[pl.ds(off + j * LANES, LANES)]
                         for j in range(nj))

        accs = lax.fori_loop(dg, dg + ng, g_body, zeros)

        def edge_body(buf):
            def body(i, accs):
                off = i * F
                new = []
                for j in range(nj):
                    x = buf[pl.ds(off + j * LANES, LANES)]
                    t = effs[j] * x
                    y = jnp.where(t > 0.0, t, jnp.exp(t) - 1.0)
                    new.append(accs[j] + y)
                return tuple(new)
            return body

        dl = s - jnp.minimum(s, N - CHS)
        accs = lax.fori_loop(dl, dl + locnt, edge_body(lb_v), accs)
        dh = gb * GS - jnp.minimum(gb * GS, N - CHE)
        accs = lax.fori_loop(dh, dh + hicnt, edge_body(hb_v), accs)

        for j in range(nj):
            row_v[pl.ds(j * LANES, LANES)] = accs[j]
        pltpu.sync_copy(row_v, out_h.at[pl.ds(b * F, F)])

    @pl.when(lo < hi)
    def _():
        fire(lo, gb0_v, lb0_v, hb0_v, sem_a)

    npairs = (hi - lo + 1) >> 1

    def pair_body(kk, carry):
        k0 = lo + 2 * kk
        k1 = k0 + 1

        @pl.when(k1 < hi)
        def _():
            fire(k1, gb1_v, lb1_v, hb1_v, sem_b)

        drain_compute_write(k0, gb0_v, lb0_v, hb0_v, sem_a, k0)

        @pl.when(k0 + 2 < hi)
        def _():
            fire(k0 + 2, gb0_v, lb0_v, hb0_v, sem_a)

        @pl.when(k1 < hi)
        def _():
            drain_compute_write(k1, gb1_v, lb1_v, hb1_v, sem_b, k1)

        return carry

    lax.fori_loop(0, npairs, pair_body, 0)


def kernel(seq, graph_len, prompt1, prompt2, prompt3, w_label, w_dff, w_down):
    N, F = seq.shape
    B = graph_len.shape[0]
    NB = N // RB
    NGT = N // GS

    gl8 = jnp.concatenate(
        [graph_len.astype(jnp.int32),
         jnp.zeros((BP - B,), jnp.int32)]).reshape(8, BP // 8)

    off8, wb48, jm, eff = pl.pallas_call(
        functools.partial(_bk_body, B, N),
        grid=(1,),
        in_specs=[
            pl.BlockSpec((8, BP // 8), lambda i: (0, 0)),
            pl.BlockSpec((1, F), lambda i: (0, 0)),
            pl.BlockSpec((1, F), lambda i: (0, 0)),
            pl.BlockSpec((1, F), lambda i: (0, 0)),
            pl.BlockSpec((1, F), lambda i: (0, 0)),
            pl.BlockSpec(memory_space=pltpu.SMEM),
            pl.BlockSpec(memory_space=pltpu.SMEM),
        ],
        out_specs=[
            pl.BlockSpec((8, BP // 8), lambda i: (0, 0)),
            pl.BlockSpec((1, 48), lambda i: (0, 0)),
            pl.BlockSpec((1, 1), lambda i: (0, 0)),
            pl.BlockSpec((1, F), lambda i: (0, 0)),
        ],
        out_shape=[
            jax.ShapeDtypeStruct((8, BP // 8), jnp.int32),
            jax.ShapeDtypeStruct((1, 48), jnp.int32),
            jax.ShapeDtypeStruct((1, 1), jnp.int32),
            jax.ShapeDtypeStruct((1, F), jnp.float32),
        ],
    )(gl8, prompt1, prompt2, prompt3, w_down,
      w_label.reshape(-1), w_dff.reshape(-1))

    gact = pl.pallas_call(
        _tc_body,
        grid_spec=pltpu.PrefetchScalarGridSpec(
            num_scalar_prefetch=1,
            grid=(NB,),
            in_specs=[
                pl.BlockSpec((RB, F), lambda j, jm: (jnp.minimum(j, jm[0] - 1), 0)),
                pl.BlockSpec((1, F), lambda j, jm: (0, 0)),
            ],
            out_specs=pl.BlockSpec(
                (NGB, F), lambda j, jm: (jnp.minimum(j, jm[0] - 1), 0)),
        ),
        out_shape=jax.ShapeDtypeStruct((NGT, F), jnp.float32),
    )(jm.reshape(1), seq, eff)

    mesh = plsc.VectorSubcoreMesh(core_axis_name="c", subcore_axis_name="s",
                                  num_cores=NUM_CORES,
                                  num_subcores=NUM_SUBCORES)
    body = functools.partial(_sc_body, N, F, B, NGT)
    out_flat = pl.kernel(
        body,
        out_type=jax.ShapeDtypeStruct((B * F,), jnp.float32),
        mesh=mesh,
        scratch_types=[
            pltpu.VMEM((CHS * F,), jnp.float32),
            pltpu.VMEM((CHS * F,), jnp.float32),
            pltpu.VMEM((CHE * F,), jnp.float32),
            pltpu.VMEM((CHS * F,), jnp.float32),
            pltpu.VMEM((CHS * F,), jnp.float32),
            pltpu.VMEM((CHE * F,), jnp.float32),
            pltpu.VMEM((BP,), jnp.int32),
            pltpu.VMEM((48,), jnp.int32),
            pltpu.VMEM((F,), jnp.float32),
            pltpu.VMEM((F,), jnp.float32),
            pltpu.SemaphoreType.DMA,
            pltpu.SemaphoreType.DMA,
        ],
    )(seq.reshape(-1), gact.reshape(-1), off8.reshape(-1), wb48.reshape(-1),
      eff.reshape(-1))
    return out_flat.reshape(B, F)


# speedup vs baseline: 1.4477x; 1.0166x over previous
import functools

import jax
import jax.numpy as jnp
from jax import lax
from jax.experimental import pallas as pl
from jax.experimental.pallas import tpu as pltpu
from jax.experimental.pallas import tpu_sc as plsc

NUM_CORES = 2
NUM_SUBCORES = 16
NUM_WORKERS = NUM_CORES * NUM_SUBCORES
LANES = 16

GS = 16
RB = 16000
NGB = RB // GS
CHS = 32
CHE = 16
BP = 1024


def _bk_body(B, N, gl_ref, p1_ref, p2_ref, p3_ref, wdn_ref, wlab_ref,
             wdff_ref, off_ref, wb_ref, jm_ref, eff_ref):
    gl8 = gl_ref[...]
    glf = gl8.astype(jnp.float32)

    io_r = lax.broadcasted_iota(jnp.int32, (128, 128), 0)
    io_c = lax.broadcasted_iota(jnp.int32, (128, 128), 1)
    tri = (io_r <= io_c).astype(jnp.float32)
    s1 = jnp.dot(glf, tri,
                 precision=lax.Precision.HIGHEST)
    rowtot = s1[:, 127:128]

    def shift(x, k):
        return jnp.concatenate(
            [jnp.zeros((k, 1), jnp.float32), x[:8 - k, :]], axis=0)

    s = rowtot
    s = s + shift(s, 1)
    s = s + shift(s, 2)
    s = s + shift(s, 4)
    rowpre = s - rowtot

    off_i = (s1 + rowpre).astype(jnp.int32)
    off_ref[...] = off_i

    ends8 = jnp.minimum(off_i, N)
    totalr = jnp.max(ends8)

    lane48 = lax.broadcasted_iota(jnp.int32, (1, 48), 1)
    acc = jnp.where(lane48 == NUM_WORKERS, B, 0)
    for w in range(1, NUM_WORKERS):
        tw = (w * totalr) >> 5
        cnt = jnp.sum((ends8 < tw).astype(jnp.int32))
        acc = acc + jnp.where(lane48 == w, cnt, 0)
    wb_ref[...] = acc

    jm = jnp.maximum((totalr + RB - 1) // RB, 1)
    jm_ref[...] = jnp.reshape(jm, (1, 1))

    wl0 = wlab_ref[0]
    wl1 = wlab_ref[1]
    wl2 = wlab_ref[2]
    wd0 = wdff_ref[0]
    wd1 = wdff_ref[1]
    eff_ref[...] = (wd0 * (1.0 + wl0 * p1_ref[...] + wl1 * p2_ref[...]
                           + wl2 * p3_ref[...]) + wd1 * wdn_ref[...])


def _tc_body(jmax, seq_ref, eff_ref, gact_ref):
    @pl.when(pl.program_id(0) < jmax[0])
    def _():
        t = eff_ref[...] * seq_ref[...]
        act = jnp.where(t > 0.0, t, jnp.exp(t) - 1.0)
        gact_ref[...] = act.reshape(NGB, GS, act.shape[1]).sum(axis=1)


def _sc_body(N, F, B, NGT, seq_h, gact_h, off_h, wb_h, eff_h,
             out_h, gb0_v, lb0_v, hb0_v, gb1_v, lb1_v, hb1_v,
             off_v, wbv_v, eff_v, row_v, sem_a, sem_b):
    nj = F // LANES
    CW = CHS * F

    cid = lax.axis_index("c")
    sid = lax.axis_index("s")
    wid = sid * NUM_CORES + cid

    pltpu.sync_copy(off_h, off_v)
    pltpu.sync_copy(wb_h, wbv_v)
    pltpu.sync_copy(eff_h, eff_v)

    effs = tuple(eff_v[pl.ds(j * LANES, LANES)] for j in range(nj))
    io16 = jnp.arange(LANES, dtype=jnp.int32)

    def pick(v, i):
        r = v[8]
        for q in range(7, -1, -1):
            r = jnp.where(i == q, v[q], r)
        return r

    wa = (wid >> 3) << 3
    vw = wbv_v[pl.ds(wa, LANES)]
    lo = pick(vw, wid - wa)
    hi = pick(vw, wid + 1 - wa)

    def seg_params(k):
        km = jnp.maximum(k - 1, 0)
        a = (km >> 3) << 3
        v = off_v[pl.ds(a, LANES)]
        sp = jnp.where(k == 0, 0, pick(v, km - a))
        s = jnp.minimum(sp, N)
        e = jnp.minimum(pick(v, k - a), N)
        ln = e - s
        ga = (s + (GS - 1)) >> 4
        gb = e >> 4
        ng = jnp.maximum(gb - ga, 0)
        locnt = jnp.where(gb > ga, ga * GS - s, ln)
        hicnt = jnp.where(gb > ga, e - gb * GS, 0)
        return s, e, ga, gb, ng, locnt, hicnt

    HW = CHE * F

    def fire(k, gb_v, lb_v, hb_v, sem):
        s, e, ga, gb, ng, locnt, hicnt = seg_params(k)

        ag = jnp.minimum(ga, NGT - CHS)
        dg = ga - ag

        @pl.when(ng > 0)
        def _():
            pltpu.async_copy(gact_h.at[pl.ds(ag * F, HW)],
                             gb_v.at[pl.ds(0, HW)], sem)

        @pl.when(dg + ng > CHE)
        def _():
            pltpu.async_copy(gact_h.at[pl.ds((ag + CHE) * F, HW)],
                             gb_v.at[pl.ds(HW, HW)], sem)

        al = jnp.minimum(s, N - CHS)
        dl = s - al

        @pl.when(locnt > 0)
        def _():
            pltpu.async_copy(seq_h.at[pl.ds(al * F, HW)],
                             lb_v.at[pl.ds(0, HW)], sem)

        @pl.when(dl + locnt > CHE)
        def _():
            pltpu.async_copy(seq_h.at[pl.ds((al + CHE) * F, HW)],
                             lb_v.at[pl.ds(HW, HW)], sem)

        @pl.when(hicnt > 0)
        def _():
            ah = jnp.minimum(gb * GS, N - CHE)
            pltpu.async_copy(seq_h.at[pl.ds(ah * F, CHE * F)], hb_v, sem)

    def drain_compute_write(k, gb_v, lb_v, hb_v, sem, b):
        s, e, ga, gb, ng, locnt, hicnt = seg_params(k)

        dg = ga - jnp.minimum(ga, NGT - CHS)
        dl0 = s - jnp.minimum(s, N - CHS)

        @pl.when(ng > 0)
        def _():
            pltpu.make_async_copy(gact_h.at[pl.ds(0, HW)],
                                  gb_v.at[pl.ds(0, HW)], sem).wait()

        @pl.when(dg + ng > CHE)
        def _():
            pltpu.make_async_copy(gact_h.at[pl.ds(0, HW)],
                                  gb_v.at[pl.ds(HW, HW)], sem).wait()

        @pl.when(locnt > 0)
        def _():
            pltpu.make_async_copy(seq_h.at[pl.ds(0, HW)],
                                  lb_v.at[pl.ds(0, HW)], sem).wait()

        @pl.when(dl0 + locnt > CHE)
        def _():
            pltpu.make_async_copy(seq_h.at[pl.ds(0, HW)],
                                  lb_v.at[pl.ds(HW, HW)], sem).wait()

        @pl.when(hicnt > 0)
        def _():
            pltpu.make_async_copy(seq_h.at[pl.ds(0, CHE * F)], hb_v,
                                  sem).wait()

        zeros = tuple(jnp.zeros((LANES,), jnp.float32) for _ in range(nj))


        def g_body(i, accs):
            off = i * F
            return tuple(accs[j] + gb_v[pl.ds(off + j * LANES, LANES)]
                         for j in range(nj))

        accs = lax.fori_loop(dg, dg + ng, g_body, zeros)

        def edge_body(buf):
            def body(i, accs):
                off = i * F
                new = []
                for j in range(nj):
                    x = buf[pl.ds(off + j * LANES, LANES)]
                    t = effs[j] * x
                    y = jnp.where(t > 0.0, t, jnp.exp(t) - 1.0)
                    new.append(accs[j] + y)
                return tuple(new)
            return body

        dl = s - jnp.minimum(s, N - CHS)
        accs = lax.fori_loop(dl, dl + locnt, edge_body(lb_v), accs)
        dh = gb * GS - jnp.minimum(gb * GS, N - CHE)
        accs = lax.fori_loop(dh, dh + hicnt, edge_body(hb_v), accs)

        for j in range(nj):
            row_v[pl.ds(j * LANES, LANES)] = accs[j]
        pltpu.sync_copy(row_v, out_h.at[pl.ds(b * F, F)])

    @pl.when(lo < hi)
    def _():
        fire(lo, gb0_v, lb0_v, hb0_v, sem_a)

    npairs = (hi - lo + 1) >> 1

    def pair_body(kk, carry):
        k0 = lo + 2 * kk
        k1 = k0 + 1

        @pl.when(k1 < hi)
        def _():
            fire(k1, gb1_v, lb1_v, hb1_v, sem_b)

        drain_compute_write(k0, gb0_v, lb0_v, hb0_v, sem_a, k0)

        @pl.when(k0 + 2 < hi)
        def _():
            fire(k0 + 2, gb0_v, lb0_v, hb0_v, sem_a)

        @pl.when(k1 < hi)
        def _():
            drain_compute_write(k1, gb1_v, lb1_v, hb1_v, sem_b, k1)

        return carry

    lax.fori_loop(0, npairs, pair_body, 0)


def kernel(seq, graph_len, prompt1, prompt2, prompt3, w_label, w_dff, w_down):
    N, F = seq.shape
    B = graph_len.shape[0]
    NB = N // RB
    NGT = N // GS

    gl8 = jnp.concatenate(
        [graph_len.astype(jnp.int32),
         jnp.zeros((BP - B,), jnp.int32)]).reshape(8, BP // 8)

    off8, wb48, jm, eff = pl.pallas_call(
        functools.partial(_bk_body, B, N),
        grid=(1,),
        in_specs=[
            pl.BlockSpec((8, BP // 8), lambda i: (0, 0)),
            pl.BlockSpec((1, F), lambda i: (0, 0)),
            pl.BlockSpec((1, F), lambda i: (0, 0)),
            pl.BlockSpec((1, F), lambda i: (0, 0)),
            pl.BlockSpec((1, F), lambda i: (0, 0)),
            pl.BlockSpec(memory_space=pltpu.SMEM),
            pl.BlockSpec(memory_space=pltpu.SMEM),
        ],
        out_specs=[
            pl.BlockSpec((8, BP // 8), lambda i: (0, 0)),
            pl.BlockSpec((1, 48), lambda i: (0, 0)),
            pl.BlockSpec((1, 1), lambda i: (0, 0)),
            pl.BlockSpec((1, F), lambda i: (0, 0)),
        ],
        out_shape=[
            jax.ShapeDtypeStruct((8, BP // 8), jnp.int32),
            jax.ShapeDtypeStruct((1, 48), jnp.int32),
            jax.ShapeDtypeStruct((1, 1), jnp.int32),
            jax.ShapeDtypeStruct((1, F), jnp.float32),
        ],
    )(gl8, prompt1, prompt2, prompt3, w_down,
      w_label.reshape(-1), w_dff.reshape(-1))

    gact = pl.pallas_call(
        _tc_body,
        grid_spec=pltpu.PrefetchScalarGridSpec(
            num_scalar_prefetch=1,
            grid=(NB,),
            in_specs=[
                pl.BlockSpec((RB, F), lambda j, jm: (jnp.minimum(j, jm[0] - 1), 0)),
                pl.BlockSpec((1, F), lambda j, jm: (0, 0)),
            ],
            out_specs=pl.BlockSpec(
                (NGB, F), lambda j, jm: (jnp.minimum(j, jm[0] - 1), 0)),
        ),
        out_shape=jax.ShapeDtypeStruct((NGT, F), jnp.float32),
    )(jm.reshape(1), seq, eff)

    mesh = plsc.VectorSubcoreMesh(core_axis_name="c", subcore_axis_name="s",
                                  num_cores=NUM_CORES,
                                  num_subcores=NUM_SUBCORES)
    body = functools.partial(_sc_body, N, F, B, NGT)
    out_flat = pl.kernel(
        body,
        out_type=jax.ShapeDtypeStruct((B * F,), jnp.float32),
        mesh=mesh,
        scratch_types=[
            pltpu.VMEM((CHS * F,), jnp.float32),
            pltpu.VMEM((CHS * F,), jnp.float32),
            pltpu.VMEM((CHE * F,), jnp.float32),
            pltpu.VMEM((CHS * F,), jnp.float32),
            pltpu.VMEM((CHS * F,), jnp.float32),
            pltpu.VMEM((CHE * F,), jnp.float32),
            pltpu.VMEM((BP,), jnp.int32),
            pltpu.VMEM((48,), jnp.int32),
            pltpu.VMEM((F,), jnp.float32),
            pltpu.VMEM((F,), jnp.float32),
            pltpu.SemaphoreType.DMA,
            pltpu.SemaphoreType.DMA,
        ],
    )(seq.reshape(-1), gact.reshape(-1), off8.reshape(-1), wb48.reshape(-1),
      eff.reshape(-1))
    return out_flat.reshape(B, F)


# speedup vs baseline: 1.4781x; 1.0210x over previous
import functools

import jax
import jax.numpy as jnp
from jax import lax
from jax.experimental import pallas as pl
from jax.experimental.pallas import tpu as pltpu
from jax.experimental.pallas import tpu_sc as plsc

NUM_CORES = 2
NUM_SUBCORES = 16
NUM_WORKERS = NUM_CORES * NUM_SUBCORES
LANES = 16

GS = 16
RB = 32000
NGB = RB // GS
CHS = 32
CHE = 16
BP = 1024


def _bk_body(B, N, gl_ref, p1_ref, p2_ref, p3_ref, wdn_ref, wlab_ref,
             wdff_ref, off_ref, wb_ref, jm_ref, eff_ref):
    gl8 = gl_ref[...]
    glf = gl8.astype(jnp.float32)

    io_r = lax.broadcasted_iota(jnp.int32, (128, 128), 0)
    io_c = lax.broadcasted_iota(jnp.int32, (128, 128), 1)
    tri = (io_r <= io_c).astype(jnp.float32)
    s1 = jnp.dot(glf, tri,
                 precision=lax.Precision.HIGHEST)
    rowtot = s1[:, 127:128]

    def shift(x, k):
        return jnp.concatenate(
            [jnp.zeros((k, 1), jnp.float32), x[:8 - k, :]], axis=0)

    s = rowtot
    s = s + shift(s, 1)
    s = s + shift(s, 2)
    s = s + shift(s, 4)
    rowpre = s - rowtot

    off_i = (s1 + rowpre).astype(jnp.int32)
    off_ref[...] = off_i

    ends8 = jnp.minimum(off_i, N)
    totalr = jnp.max(ends8)

    lane48 = lax.broadcasted_iota(jnp.int32, (1, 48), 1)
    acc = jnp.where(lane48 == NUM_WORKERS, B, 0)
    for w in range(1, NUM_WORKERS):
        tw = (w * totalr) >> 5
        cnt = jnp.sum((ends8 < tw).astype(jnp.int32))
        acc = acc + jnp.where(lane48 == w, cnt, 0)
    wb_ref[...] = acc

    jm = jnp.maximum((totalr + RB - 1) // RB, 1)
    jm_ref[...] = jnp.reshape(jm, (1, 1))

    wl0 = wlab_ref[0]
    wl1 = wlab_ref[1]
    wl2 = wlab_ref[2]
    wd0 = wdff_ref[0]
    wd1 = wdff_ref[1]
    eff_ref[...] = (wd0 * (1.0 + wl0 * p1_ref[...] + wl1 * p2_ref[...]
                           + wl2 * p3_ref[...]) + wd1 * wdn_ref[...])


def _tc_body(jmax, seq_ref, eff_ref, gact_ref):
    @pl.when(pl.program_id(0) < jmax[0])
    def _():
        t = eff_ref[...] * seq_ref[...]
        act = jnp.where(t > 0.0, t, jnp.exp(t) - 1.0)
        gact_ref[...] = act.reshape(NGB, GS, act.shape[1]).sum(axis=1)


def _sc_body(N, F, B, NGT, seq_h, gact_h, off_h, wb_h, eff_h,
             out_h, gb0_v, lb0_v, hb0_v, gb1_v, lb1_v, hb1_v,
             off_v, wbv_v, eff_v, row_v, sem_a, sem_b):
    nj = F // LANES
    CW = CHS * F

    cid = lax.axis_index("c")
    sid = lax.axis_index("s")
    wid = sid * NUM_CORES + cid

    pltpu.sync_copy(off_h, off_v)
    pltpu.sync_copy(wb_h, wbv_v)
    pltpu.sync_copy(eff_h, eff_v)

    effs = tuple(eff_v[pl.ds(j * LANES, LANES)] for j in range(nj))
    io16 = jnp.arange(LANES, dtype=jnp.int32)

    def pick(v, i):
        r = v[8]
        for q in range(7, -1, -1):
            r = jnp.where(i == q, v[q], r)
        return r

    wa = (wid >> 3) << 3
    vw = wbv_v[pl.ds(wa, LANES)]
    lo = pick(vw, wid - wa)
    hi = pick(vw, wid + 1 - wa)

    def seg_params(k):
        km = jnp.maximum(k - 1, 0)
        a = (km >> 3) << 3
        v = off_v[pl.ds(a, LANES)]
        sp = jnp.where(k == 0, 0, pick(v, km - a))
        s = jnp.minimum(sp, N)
        e = jnp.minimum(pick(v, k - a), N)
        ln = e - s
        ga = (s + (GS - 1)) >> 4
        gb = e >> 4
        ng = jnp.maximum(gb - ga, 0)
        locnt = jnp.where(gb > ga, ga * GS - s, ln)
        hicnt = jnp.where(gb > ga, e - gb * GS, 0)
        return s, e, ga, gb, ng, locnt, hicnt

    HW = CHE * F

    def fire(k, gb_v, lb_v, hb_v, sem):
        s, e, ga, gb, ng, locnt, hicnt = seg_params(k)

        ag = jnp.minimum(ga, NGT - CHS)
        dg = ga - ag

        @pl.when(ng > 0)
        def _():
            pltpu.async_copy(gact_h.at[pl.ds(ag * F, HW)],
                             gb_v.at[pl.ds(0, HW)], sem)

        @pl.when(dg + ng > CHE)
        def _():
            pltpu.async_copy(gact_h.at[pl.ds((ag + CHE) * F, HW)],
                             gb_v.at[pl.ds(HW, HW)], sem)

        al = jnp.minimum(s, N - CHS)
        dl = s - al

        @pl.when(locnt > 0)
        def _():
            pltpu.async_copy(seq_h.at[pl.ds(al * F, HW)],
                             lb_v.at[pl.ds(0, HW)], sem)

        @pl.when(dl + locnt > CHE)
        def _():
            pltpu.async_copy(seq_h.at[pl.ds((al + CHE) * F, HW)],
                             lb_v.at[pl.ds(HW, HW)], sem)

        @pl.when(hicnt > 0)
        def _():
            ah = jnp.minimum(gb * GS, N - CHE)
            pltpu.async_copy(seq_h.at[pl.ds(ah * F, CHE * F)], hb_v, sem)

    def drain_compute_write(k, gb_v, lb_v, hb_v, sem, b):
        s, e, ga, gb, ng, locnt, hicnt = seg_params(k)

        dg = ga - jnp.minimum(ga, NGT - CHS)
        dl0 = s - jnp.minimum(s, N - CHS)

        @pl.when(ng > 0)
        def _():
            pltpu.make_async_copy(gact_h.at[pl.ds(0, HW)],
                                  gb_v.at[pl.ds(0, HW)], sem).wait()

        @pl.when(dg + ng > CHE)
        def _():
            pltpu.make_async_copy(gact_h.at[pl.ds(0, HW)],
                                  gb_v.at[pl.ds(HW, HW)], sem).wait()

        @pl.when(locnt > 0)
        def _():
            pltpu.make_async_copy(seq_h.at[pl.ds(0, HW)],
                                  lb_v.at[pl.ds(0, HW)], sem).wait()

        @pl.when(dl0 + locnt > CHE)
        def _():
            pltpu.make_async_copy(seq_h.at[pl.ds(0, HW)],
                                  lb_v.at[pl.ds(HW, HW)], sem).wait()

        @pl.when(hicnt > 0)
        def _():
            pltpu.make_async_copy(seq_h.at[pl.ds(0, CHE * F)], hb_v,
                                  sem).wait()

        zeros = tuple(jnp.zeros((LANES,), jnp.float32) for _ in range(nj))


        def g_body(i, accs):
            off = i * F
            return tuple(accs[j] + gb_v[pl.ds(off + j * LANES, LANES)]
                         for j in range(nj))

        accs = lax.fori_loop(dg, dg + ng, g_body, zeros)

        def edge_body(buf):
            def body(i, accs):
                off = i * F
                new = []
                for j in range(nj):
                    x = buf[pl.ds(off + j * LANES, LANES)]
                    t = effs[j] * x
                    y = jnp.where(t > 0.0, t, jnp.exp(t) - 1.0)
                    new.append(accs[j] + y)
                return tuple(new)
            return body

        dl = s - jnp.minimum(s, N - CHS)
        accs = lax.fori_loop(dl, dl + locnt, edge_body(lb_v), accs)
        dh = gb * GS - jnp.minimum(gb * GS, N - CHE)
        accs = lax.fori_loop(dh, dh + hicnt, edge_body(hb_v), accs)

        for j in range(nj):
            row_v[pl.ds(j * LANES, LANES)] = accs[j]
        pltpu.sync_copy(row_v, out_h.at[pl.ds(b * F, F)])

    @pl.when(lo < hi)
    def _():
        fire(lo, gb0_v, lb0_v, hb0_v, sem_a)

    npairs = (hi - lo + 1) >> 1

    def pair_body(kk, carry):
        k0 = lo + 2 * kk
        k1 = k0 + 1

        @pl.when(k1 < hi)
        def _():
            fire(k1, gb1_v, lb1_v, hb1_v, sem_b)

        drain_compute_write(k0, gb0_v, lb0_v, hb0_v, sem_a, k0)

        @pl.when(k0 + 2 < hi)
        def _():
            fire(k0 + 2, gb0_v, lb0_v, hb0_v, sem_a)

        @pl.when(k1 < hi)
        def _():
            drain_compute_write(k1, gb1_v, lb1_v, hb1_v, sem_b, k1)

        return carry

    lax.fori_loop(0, npairs, pair_body, 0)


def kernel(seq, graph_len, prompt1, prompt2, prompt3, w_label, w_dff, w_down):
    N, F = seq.shape
    B = graph_len.shape[0]
    NB = N // RB
    NGT = N // GS

    gl8 = jnp.concatenate(
        [graph_len.astype(jnp.int32),
         jnp.zeros((BP - B,), jnp.int32)]).reshape(8, BP // 8)

    off8, wb48, jm, eff = pl.pallas_call(
        functools.partial(_bk_body, B, N),
        grid=(1,),
        in_specs=[
            pl.BlockSpec((8, BP // 8), lambda i: (0, 0)),
            pl.BlockSpec((1, F), lambda i: (0, 0)),
            pl.BlockSpec((1, F), lambda i: (0, 0)),
            pl.BlockSpec((1, F), lambda i: (0, 0)),
            pl.BlockSpec((1, F), lambda i: (0, 0)),
            pl.BlockSpec(memory_space=pltpu.SMEM),
            pl.BlockSpec(memory_space=pltpu.SMEM),
        ],
        out_specs=[
            pl.BlockSpec((8, BP // 8), lambda i: (0, 0)),
            pl.BlockSpec((1, 48), lambda i: (0, 0)),
            pl.BlockSpec((1, 1), lambda i: (0, 0)),
            pl.BlockSpec((1, F), lambda i: (0, 0)),
        ],
        out_shape=[
            jax.ShapeDtypeStruct((8, BP // 8), jnp.int32),
            jax.ShapeDtypeStruct((1, 48), jnp.int32),
            jax.ShapeDtypeStruct((1, 1), jnp.int32),
            jax.ShapeDtypeStruct((1, F), jnp.float32),
        ],
    )(gl8, prompt1, prompt2, prompt3, w_down,
      w_label.reshape(-1), w_dff.reshape(-1))

    gact = pl.pallas_call(
        _tc_body,
        grid_spec=pltpu.PrefetchScalarGridSpec(
            num_scalar_prefetch=1,
            grid=(NB,),
            in_specs=[
                pl.BlockSpec((RB, F), lambda j, jm: (jnp.minimum(j, jm[0] - 1), 0)),
                pl.BlockSpec((1, F), lambda j, jm: (0, 0)),
            ],
            out_specs=pl.BlockSpec(
                (NGB, F), lambda j, jm: (jnp.minimum(j, jm[0] - 1), 0)),
        ),
        out_shape=jax.ShapeDtypeStruct((NGT, F), jnp.float32),
    )(jm.reshape(1), seq, eff)

    mesh = plsc.VectorSubcoreMesh(core_axis_name="c", subcore_axis_name="s",
                                  num_cores=NUM_CORES,
                                  num_subcores=NUM_SUBCORES)
    body = functools.partial(_sc_body, N, F, B, NGT)
    out_flat = pl.kernel(
        body,
        out_type=jax.ShapeDtypeStruct((B * F,), jnp.float32),
        mesh=mesh,
        scratch_types=[
            pltpu.VMEM((CHS * F,), jnp.float32),
            pltpu.VMEM((CHS * F,), jnp.float32),
            pltpu.VMEM((CHE * F,), jnp.float32),
            pltpu.VMEM((CHS * F,), jnp.float32),
            pltpu.VMEM((CHS * F,), jnp.float32),
            pltpu.VMEM((CHE * F,), jnp.float32),
            pltpu.VMEM((BP,), jnp.int32),
            pltpu.VMEM((48,), jnp.int32),
            pltpu.VMEM((F,), jnp.float32),
            pltpu.VMEM((F,), jnp.float32),
            pltpu.SemaphoreType.DMA,
            pltpu.SemaphoreType.DMA,
        ],
    )(seq.reshape(-1), gact.reshape(-1), off8.reshape(-1), wb48.reshape(-1),
      eff.reshape(-1))
    return out_flat.reshape(B, F)
